# Initial kernel scaffold; baseline (speedup 1.0000x reference)
import jax
import jax.numpy as jnp
from jax.experimental import pallas as pl


def kernel(x, W_lin, b_lin, w_gate, training):
    raise NotImplementedError("write your pallas kernel here")


# speedup vs baseline: 1.3077x; 1.3077x over previous
import functools

import jax
import jax.numpy as jnp
import numpy as np
from jax.experimental import pallas as pl
from jax.experimental.pallas import tpu as pltpu
from jax.experimental.pallas import tpu_sc as plsc

SEQ = 32
NFREQ = 16
NSEG = 9
BATCH = 16
BT = 512
C = 512
HW = 256
D = 128
K = HW * D

HWB = 128
NHW = HW // HWB
CBLK = 64
NCB = C // CBLK
NSTEP = NHW * NCB


def _build_dft() -> np.ndarray:
    t = np.arange(SEQ)
    f = np.arange(1, NFREQ + 1)
    ang = 2.0 * np.pi * np.outer(f, t) / SEQ
    cos = np.cos(ang) / np.sqrt(SEQ)
    msin = -np.sin(ang) / np.sqrt(SEQ)
    eye = np.eye(BATCH)
    fr = np.kron(eye, cos)
    fi = np.kron(eye, msin)
    return np.concatenate([fr, fi], axis=0).astype(np.float32)


_FCOMB = _build_dft()


def _w_copy(w_hbm, wbuf, sems, jc, jhw, slot):
    return pltpu.make_async_copy(
        w_hbm.at[pl.ds(jc * CBLK, CBLK), :, pl.ds(jhw * HWB, HWB)],
        wbuf.at[slot], sems.at[slot])


def _x_copy(x_hbm, xbuf, semx, jhw):
    return pltpu.make_async_copy(
        x_hbm.at[:, pl.ds(jhw * HWB, HWB), :], xbuf, semx)


def _tc_body(w_hbm, x_hbm, f_ref, amp_ref, xbuf, wbuf, wt, hmat, semx, sems):
    j = pl.program_id(0)
    jhw = j // NCB
    jc = jax.lax.rem(j, NCB)
    slot = jax.lax.rem(j, 2)

    @pl.when(jc == 0)
    def _():
        _x_copy(x_hbm, xbuf, semx, jhw).start()

    @pl.when(j == 0)
    def _():
        _w_copy(w_hbm, wbuf, sems, jc, jhw, slot).start()

    @pl.when(j + 1 < NSTEP)
    def _():
        jn = j + 1
        _w_copy(w_hbm, wbuf, sems, jax.lax.rem(jn, NCB), jn // NCB,
                1 - slot).start()

    @pl.when(jc == 0)
    def _():
        _x_copy(x_hbm, xbuf, semx, jhw).wait()

    _w_copy(w_hbm, wbuf, sems, jc, jhw, slot).wait()

    xb = xbuf[...].reshape(BT, HWB * D)
    row = jax.lax.broadcasted_iota(jnp.int32, (HWB, HWB), 0)
    col = jax.lax.broadcasted_iota(jnp.int32, (HWB, HWB), 1)
    eye = jnp.where(row == col, jnp.float32(1.0), jnp.float32(0.0))
    for cc in range(CBLK):
        wt[cc, :, :] = jax.lax.dot_general(
            eye, wbuf[slot, cc, :, :], (((1,), (1,)), ((), ())),
            preferred_element_type=jnp.float32)
    wb = wt[...].reshape(CBLK, HWB * D)
    part_t = jax.lax.dot_general(
        wb, xb, (((1,), (1,)), ((), ())), preferred_element_type=jnp.float32)

    @pl.when(jhw == 0)
    def _():
        hmat[pl.ds(jc * CBLK, CBLK), :] = part_t

    @pl.when(jhw > 0)
    def _():
        hmat[pl.ds(jc * CBLK, CBLK), :] += part_t

    @pl.when(j == NSTEP - 1)
    def _():
        res = jax.lax.dot_general(
            f_ref[...], hmat[...], (((1,), (1,)), ((), ())),
            preferred_element_type=jnp.float32)
        re = res[:BATCH * NFREQ, :]
        im = res[BATCH * NFREQ:, :]
        mag = jnp.sqrt(re * re + im * im)
        amp_ref[...] = jnp.mean(mag, axis=1, keepdims=True)


def _tc_amp(x3, w3, fcomb, interpret=False):
    return pl.pallas_call(
        _tc_body,
        grid=(NSTEP,),
        in_specs=[
            pl.BlockSpec(memory_space=pl.ANY),
            pl.BlockSpec(memory_space=pl.ANY),
            pl.BlockSpec((BATCH * NFREQ * 2, BT), lambda j: (0, 0)),
        ],
        out_specs=pl.BlockSpec((BATCH * NFREQ, 1), lambda j: (0, 0)),
        out_shape=jax.ShapeDtypeStruct((BATCH * NFREQ, 1), jnp.float32),
        scratch_shapes=[
            pltpu.VMEM((BT, HWB, D), jnp.float32),
            pltpu.VMEM((2, CBLK, D, HWB), jnp.float32),
            pltpu.VMEM((CBLK, HWB, D), jnp.float32),
            pltpu.VMEM((C, BT), jnp.float32),
            pltpu.SemaphoreType.DMA,
            pltpu.SemaphoreType.DMA((2,)),
        ],
        interpret=interpret,
    )(w3, x3, fcomb)


def _sc_gate_body(amp_hbm, wgt_hbm, gates_hbm, load_hbm,
                  amp_v, wgt_v, gates_v, load_v):
    cid = jax.lax.axis_index("c")
    sid = jax.lax.axis_index("s")

    @pl.when(jnp.logical_and(cid == 0, sid == 0))
    def _():
        pltpu.sync_copy(amp_hbm, amp_v)
        pltpu.sync_copy(wgt_hbm, wgt_v)
        iota = jax.lax.iota(jnp.int32, 16)

        def shuf(v, sh):
            return v.at[iota ^ sh].get(mode="promise_in_bounds")

        def lane_sum(v):
            for sh in (8, 4, 2, 1):
                v = v + shuf(v, sh)
            return v

        def lane_max(v):
            for sh in (8, 4, 2, 1):
                v = jnp.maximum(v, shuf(v, sh))
            return v

        def lane_min(v):
            for sh in (8, 4, 2, 1):
                v = jnp.minimum(v, shuf(v, sh))
            return v

        ninf = jnp.float32(-3e38)
        pad = jnp.where(iota < NSEG, jnp.float32(0.0), ninf)
        counts = jnp.where(iota < 0, 1, 0)
        for b in range(BATCH):
            ab = amp_v[pl.ds(b * NFREQ, 16)]
            lvec = pad
            for s in range(NSEG):
                ls = lane_sum(ab * wgt_v[s, :])
                lvec = jnp.where(iota == s, ls, lvec)
            m1 = lane_max(lvec)
            j1 = lane_min(jnp.where(lvec == m1, iota, jnp.int32(16)))
            sel1 = iota == j1
            lvec2 = jnp.where(sel1, ninf, lvec)
            m2 = lane_max(lvec2)
            j2 = lane_min(jnp.where(lvec2 == m2, iota, jnp.int32(16)))
            sel2 = iota == j2
            t = jnp.exp(m2 - m1)
            g1 = 1.0 / (1.0 + t)
            g2 = t / (1.0 + t)
            gates_v[b, :] = jnp.where(sel1, g1, 0.0) + jnp.where(sel2, g2, 0.0)
            counts = counts + jnp.where(sel1, 1, 0) + jnp.where(sel2, 1, 0)
        load_v[...] = counts
        pltpu.sync_copy(gates_v, gates_hbm)
        pltpu.sync_copy(load_v, load_hbm)


@functools.cache
def _sc_gate():
    return pl.kernel(
        _sc_gate_body,
        mesh=plsc.VectorSubcoreMesh(core_axis_name="c", subcore_axis_name="s"),
        out_type=[
            jax.ShapeDtypeStruct((BATCH, 16), jnp.float32),
            jax.ShapeDtypeStruct((16,), jnp.int32),
        ],
        scratch_types=[
            pltpu.VMEM((BATCH * NFREQ,), jnp.float32),
            pltpu.VMEM((16, 16), jnp.float32),
            pltpu.VMEM((BATCH, 16), jnp.float32),
            pltpu.VMEM((16,), jnp.int32),
        ],
    )


def kernel(x, W_lin, b_lin, w_gate, training):
    del b_lin, training
    x3 = x.reshape(BT, HW, D)
    w3 = W_lin.reshape(C, D, HW)
    amp = _tc_amp(x3, w3, jnp.asarray(_FCOMB)).reshape(BATCH * NFREQ)
    wgt_pad = jnp.zeros((16, 16), jnp.float32).at[:NSEG, :].set(w_gate.T)
    gates_pad, load_pad = _sc_gate()(amp, wgt_pad)
    return gates_pad[:, :NSEG], load_pad[:NSEG]
